# baseline (speedup 1.0000x reference)
import functools
import math

import jax
import jax.numpy as jnp
from jax import lax
from jax.experimental import pallas as pl
from jax.experimental.pallas import tpu as pltpu
from jax.experimental.pallas import tpu_sc as plsc

_NC = 2
_NS = 16
_NW = _NC * _NS

_CHUNK = 1024


def _emb_kernel(b_per_w, n_chunks, d, scale,
                idx_hbm, table_hbm, out_hbm, idx_v, rows_v, sem):
    wid = lax.axis_index("s") * _NC + lax.axis_index("c")
    base = wid * b_per_w

    def chunk_body(g, _):
        off = base + g * _CHUNK
        pltpu.sync_copy(idx_hbm.at[pl.ds(off, _CHUNK)], idx_v)
        pltpu.async_copy(table_hbm.at[idx_v], rows_v, sem).wait()

        def row_body(i, _):
            for h in range(0, d, 16):
                rows_v[i, pl.ds(h, 16)] = rows_v[i, pl.ds(h, 16)] * scale
            return 0

        lax.fori_loop(0, _CHUNK, row_body, 0)
        pltpu.sync_copy(rows_v, out_hbm.at[pl.ds(off, _CHUNK)])
        return 0

    lax.fori_loop(0, n_chunks, chunk_body, 0)


def kernel(tokens, table):
    s, t = tokens.shape
    v, d = table.shape
    b = s * t
    assert b % (_NW * _CHUNK) == 0
    b_per_w = b // _NW
    n_chunks = b_per_w // _CHUNK
    scale = float(math.sqrt(d))

    idx = tokens.reshape(b).astype(jnp.int32)

    mesh = plsc.VectorSubcoreMesh(core_axis_name="c", subcore_axis_name="s")
    run = pl.kernel(
        functools.partial(_emb_kernel, b_per_w, n_chunks, d, scale),
        mesh=mesh,
        out_type=jax.ShapeDtypeStruct((b, d), jnp.float32),
        scratch_types=[
            pltpu.VMEM((_CHUNK,), jnp.int32),
            pltpu.VMEM((_CHUNK, d), jnp.float32),
            pltpu.SemaphoreType.DMA,
        ],
        compiler_params=pltpu.CompilerParams(use_tc_tiling_on_sc=False),
    )
    out = run(idx, table)
    return out.reshape(s, t, d)


# speedup vs baseline: 1.0861x; 1.0861x over previous
import functools
import math

import jax
import jax.numpy as jnp
from jax import lax
from jax.experimental import pallas as pl
from jax.experimental.pallas import tpu as pltpu
from jax.experimental.pallas import tpu_sc as plsc

_NC = 2
_NS = 16
_NW = _NC * _NS

_NBUF = 4
_CHUNK = 640


def _emb_kernel(b_per_w, n_chunks, d, scale,
                idx_hbm, table_hbm, out_hbm, idx_all, rows, gsem, wsem):
    wid = lax.axis_index("s") * _NC + lax.axis_index("c")
    base = wid * b_per_w
    pltpu.sync_copy(idx_hbm.at[pl.ds(base, b_per_w)], idx_all)

    def gather_start(k, b):
        pltpu.async_copy(
            table_hbm.at[idx_all.at[pl.ds(k * _CHUNK, _CHUNK)]], rows[b],
            gsem[b])

    def gather_wait(b):
        pltpu.make_async_copy(
            table_hbm.at[idx_all.at[pl.ds(0, _CHUNK)]], rows[b],
            gsem[b]).wait()

    def wb_start(k, b):
        pltpu.async_copy(
            rows[b], out_hbm.at[pl.ds(base + k * _CHUNK, _CHUNK)], wsem[b])

    def wb_wait(b):
        pltpu.make_async_copy(
            rows[b], out_hbm.at[pl.ds(base, _CHUNK)], wsem[b]).wait()

    def scale_rows(b):
        rb = rows[b]

        @plsc.parallel_loop(0, _CHUNK, 1, unroll=8)
        def _(i):
            for h in range(0, d, 16):
                rb[i, pl.ds(h, 16)] = rb[i, pl.ds(h, 16)] * scale

    for b in range(_NBUF - 1):
        gather_start(b, b)

    def group(q, _):
        for b in range(_NBUF):
            k = q * _NBUF + b
            gather_wait(b)
            scale_rows(b)
            wb_start(k, b)
            k2 = k + _NBUF - 1
            pb = (b - 1) % _NBUF

            @pl.when(k2 < n_chunks)
            def _():
                if b == 0:
                    @pl.when(k > 0)
                    def _():
                        wb_wait(pb)
                else:
                    wb_wait(pb)
                gather_start(k2, pb)
        return 0

    lax.fori_loop(0, n_chunks // _NBUF, group, 0)

    for b in range(_NBUF):
        wb_wait(b)


def kernel(tokens, table):
    s, t = tokens.shape
    v, d = table.shape
    b = s * t
    assert b % (_NW * _NBUF * _CHUNK) == 0
    b_per_w = b // _NW
    n_chunks = b_per_w // _CHUNK
    scale = float(math.sqrt(d))

    idx = tokens.reshape(b).astype(jnp.int32)

    mesh = plsc.VectorSubcoreMesh(core_axis_name="c", subcore_axis_name="s")
    run = pl.kernel(
        functools.partial(_emb_kernel, b_per_w, n_chunks, d, scale),
        mesh=mesh,
        out_type=jax.ShapeDtypeStruct((b, d), jnp.float32),
        scratch_types=[
            pltpu.VMEM((b_per_w,), jnp.int32),
            [pltpu.VMEM((_CHUNK, d), jnp.float32) for _ in range(_NBUF)],
            [pltpu.SemaphoreType.DMA for _ in range(_NBUF)],
            [pltpu.SemaphoreType.DMA for _ in range(_NBUF)],
        ],
        compiler_params=pltpu.CompilerParams(use_tc_tiling_on_sc=False),
    )
    out = run(idx, table)
    return out.reshape(s, t, d)


# speedup vs baseline: 2.1579x; 1.9868x over previous
import functools
import math

import jax
import jax.numpy as jnp
from jax import lax
from jax.experimental import pallas as pl
from jax.experimental.pallas import tpu as pltpu
from jax.experimental.pallas import tpu_sc as plsc

_NC = 2
_NS = 16
_NW = _NC * _NS

_NBUF = 4
_U = 128


def _emb_kernel(units_per_w, n_jc, d, scale,
                idx_hbm, table_hbm, out_hbm, idx_all, bufs, tbufs, gsem, wsem):
    wid = lax.axis_index("s") * _NC + lax.axis_index("c")
    base_u = wid * units_per_w
    pltpu.sync_copy(idx_hbm.at[pl.ds(base_u * _U, units_per_w * _U)], idx_all)

    iota16 = lax.iota(jnp.int32, 16)
    row_ids = [l0 + iota16 for l0 in range(0, _U, 16)]

    def gather_start(uu, b):
        pltpu.async_copy(
            table_hbm.at[idx_all.at[pl.ds(uu * _U, _U)]], bufs[b], gsem[b])

    def gather_wait(b):
        pltpu.make_async_copy(
            table_hbm.at[idx_all.at[pl.ds(0, _U)]], bufs[b], gsem[b]).wait()

    def wb_start(uu, b):
        u = base_u + uu
        j = u >> 7
        c = u & 127
        pltpu.async_copy(tbufs[b], out_hbm.at[j, :, c], wsem[b])

    def wb_wait(b):
        pltpu.make_async_copy(tbufs[b], out_hbm.at[0, :, 0], wsem[b]).wait()

    def transpose_scale(b):
        buf, tbuf = bufs[b], tbufs[b]

        @plsc.parallel_loop(0, d, 1, unroll=2)
        def _(e):
            col = jnp.full((16,), 0, jnp.int32) + e
            r = e >> 3
            s = e & 7
            for k, rid in enumerate(row_ids):
                v = plsc.load_gather(buf, [rid, col])
                tbuf[r, s, pl.ds(k * 16, 16)] = v * scale

    for b in range(_NBUF - 1):
        gather_start(b, b)

    def group(q, _):
        for b in range(_NBUF):
            uu = q * _NBUF + b
            gather_wait(b)
            transpose_scale(b)
            wb_start(uu, b)
            u2 = uu + _NBUF - 1
            pb = (b - 1) % _NBUF

            @pl.when(u2 < units_per_w)
            def _():
                if b == 0:
                    @pl.when(uu > 0)
                    def _():
                        wb_wait(pb)
                else:
                    wb_wait(pb)
                gather_start(u2, pb)
        return 0

    lax.fori_loop(0, units_per_w // _NBUF, group, 0)

    for b in range(_NBUF):
        wb_wait(b)


def kernel(tokens, table):
    s, t = tokens.shape
    v, d = table.shape
    b = s * t
    n_jc = (s // _U) * t
    assert s % _U == 0 and n_jc % (_NW * _NBUF) == 0 and d == 32
    units_per_w = n_jc // _NW
    scale = float(math.sqrt(d))

    idx = tokens.T.reshape(b)

    mesh = plsc.VectorSubcoreMesh(core_axis_name="c", subcore_axis_name="s")
    run = pl.kernel(
        functools.partial(_emb_kernel, units_per_w, n_jc, d, scale),
        mesh=mesh,
        out_type=jax.ShapeDtypeStruct((t, d // 8, s // _U, 8, _U),
                                      jnp.float32),
        scratch_types=[
            pltpu.VMEM((units_per_w * _U,), jnp.int32),
            [pltpu.VMEM((_U, d), jnp.float32) for _ in range(_NBUF)],
            [pltpu.VMEM((d // 8, 8, _U), jnp.float32) for _ in range(_NBUF)],
            [pltpu.SemaphoreType.DMA for _ in range(_NBUF)],
            [pltpu.SemaphoreType.DMA for _ in range(_NBUF)],
        ],
        compiler_params=pltpu.CompilerParams(use_tc_tiling_on_sc=False,
                                             needs_layout_passes=False),
    )
    out5 = run(idx, table)
    return out5.transpose(2, 4, 0, 1, 3).reshape(s, t, d)
